# baseline (speedup 1.0000x reference)
import functools

import jax
import jax.numpy as jnp
from jax import lax
from jax.experimental import pallas as pl
from jax.experimental.pallas import tpu as pltpu
from jax.experimental.pallas import tpu_sc as plsc

D_MODEL = 64
BLK = 128
NBUF = 2


def _make_sc_gather(n_blocks, steps_per_tile, num_cores):
    mesh = plsc.VectorSubcoreMesh(core_axis_name="c", subcore_axis_name="s")
    n_groups = steps_per_tile // NBUF
    n_j = n_blocks * BLK // 16384
    tiles_i = 16384 // BLK

    @functools.partial(
        pl.kernel,
        mesh=mesh,
        out_type=jax.ShapeDtypeStruct(
            (n_j, D_MODEL // 8, tiles_i, 8, BLK), jnp.float32
        ),
        scratch_types=[
            pltpu.VMEM((steps_per_tile, BLK), jnp.int32),
            pltpu.VMEM((NBUF, BLK, D_MODEL), jnp.float32),
            pltpu.VMEM((NBUF, D_MODEL // 8, 8, BLK), jnp.float32),
            pltpu.SemaphoreType.DMA((NBUF,)),
            pltpu.SemaphoreType.DMA((NBUF,)),
        ],
        compiler_params=pltpu.CompilerParams(
            use_tc_tiling_on_sc=False, needs_layout_passes=False
        ),
    )
    def k(idx_hbm, table_hbm, out_hbm, idx_v, rows_v, blk_v, gsem, osem):
        wid = lax.axis_index("s") * num_cores + lax.axis_index("c")
        row0 = wid * steps_per_tile
        pltpu.sync_copy(idx_hbm.at[pl.ds(row0, steps_per_tile)], idx_v)

        lanes = jnp.arange(16, dtype=jnp.int32)

        def gather(t, b):
            pltpu.async_copy(table_hbm.at[idx_v.at[t]], rows_v.at[b], gsem.at[b])

        def wait_gather(b):
            pltpu.make_async_copy(
                table_hbm.at[pl.ds(0, BLK)], rows_v.at[b], gsem.at[b]
            ).wait()

        def transpose(b):
            @plsc.parallel_loop(0, D_MODEL * 8, unroll=16)
            def _(u):
                d = u // 8
                ic = u % 8
                dcol = jnp.full((16,), d, dtype=jnp.int32)
                v = plsc.load_gather(rows_v.at[b], [lanes + ic * 16, dcol])
                blk_v[b, d // 8, d % 8, pl.ds(ic * 16, 16)] = v

        def writeback(t, b):
            n = row0 + t
            j = n // tiles_i
            tc = n % tiles_i
            pltpu.async_copy(blk_v.at[b], out_hbm.at[j, :, tc], osem.at[b])

        def wait_writeback(b):
            pltpu.make_async_copy(
                blk_v.at[b], out_hbm.at[0, :, 0], osem.at[b]
            ).wait()

        for b in range(NBUF):
            gather(b, b)

        def group(gi, carry):
            t0 = gi * NBUF
            for b in range(NBUF):
                wait_gather(b)
                transpose(b)
                writeback(t0 + b, b)
            for b in range(NBUF):
                wait_writeback(b)
                gather(t0 + b + NBUF, b)
            return carry

        lax.fori_loop(0, n_groups - 1, group, 0)

        t0 = (n_groups - 1) * NBUF
        for b in range(NBUF):
            wait_gather(b)
            transpose(b)
            writeback(t0 + b, b)
        for b in range(NBUF):
            wait_writeback(b)

    return k


def kernel(attribute, table):
    b0, b1 = attribute.shape
    n = b0 * b1
    info = plsc.get_sparse_core_info()
    nw = info.num_cores * info.num_subcores
    n_blocks = n // BLK
    steps_per_tile = n_blocks // nw
    idx = attribute.T.reshape(n_blocks, BLK).astype(jnp.int32)
    out5 = _make_sc_gather(n_blocks, steps_per_tile, info.num_cores)(idx, table)
    return out5.transpose(2, 4, 0, 1, 3).reshape(b0, b1, D_MODEL)


# speedup vs baseline: 1.5982x; 1.5982x over previous
import functools

import jax
import jax.numpy as jnp
from jax import lax
from jax.experimental import pallas as pl
from jax.experimental.pallas import tpu as pltpu
from jax.experimental.pallas import tpu_sc as plsc

D_MODEL = 64
BLK = 128
NBUF = 2


def _make_sc_gather(n_blocks, steps_per_tile, num_cores):
    mesh = plsc.VectorSubcoreMesh(core_axis_name="c", subcore_axis_name="s")
    n_groups = steps_per_tile // NBUF
    n_j = n_blocks * BLK // 16384
    tiles_i = 16384 // BLK

    @functools.partial(
        pl.kernel,
        mesh=mesh,
        out_type=jax.ShapeDtypeStruct(
            (n_j, D_MODEL // 8, tiles_i, 8, BLK), jnp.float32
        ),
        scratch_types=[
            pltpu.VMEM((steps_per_tile, BLK), jnp.int32),
            pltpu.VMEM((NBUF, BLK, D_MODEL), jnp.float32),
            pltpu.VMEM((NBUF, D_MODEL // 8, 8, BLK), jnp.float32),
            pltpu.SemaphoreType.DMA((NBUF,)),
            pltpu.SemaphoreType.DMA((NBUF,)),
        ],
        compiler_params=pltpu.CompilerParams(
            use_tc_tiling_on_sc=False, needs_layout_passes=False
        ),
    )
    def k(idx_hbm, table_hbm, out_hbm, idx_v, rows_v, blk_v, gsem, osem):
        wid = lax.axis_index("s") * num_cores + lax.axis_index("c")
        row0 = wid * steps_per_tile
        pltpu.sync_copy(idx_hbm.at[pl.ds(row0, steps_per_tile)], idx_v)

        lanes = jnp.arange(16, dtype=jnp.int32)

        def gather(t, b):
            pltpu.async_copy(table_hbm.at[idx_v.at[t]], rows_v.at[b], gsem.at[b])

        def wait_gather(b):
            pltpu.make_async_copy(
                table_hbm.at[pl.ds(0, BLK)], rows_v.at[b], gsem.at[b]
            ).wait()

        def transpose(b):
            @plsc.parallel_loop(0, 32, unroll=4)
            def _(s):
                i0 = (s // 4) * 16
                d0 = (s % 4) * 16
                ivec = lanes + i0
                for k in range(16):
                    dvec = ((lanes + k) & 15) + d0
                    v = plsc.load_gather(rows_v.at[b], [ivec, dvec])
                    plsc.store_scatter(
                        blk_v.at[b], [dvec >> 3, dvec & 7, ivec], v
                    )

        def writeback(t, b):
            n = row0 + t
            j = n // tiles_i
            tc = n % tiles_i
            pltpu.async_copy(blk_v.at[b], out_hbm.at[j, :, tc], osem.at[b])

        def wait_writeback(b):
            pltpu.make_async_copy(
                blk_v.at[b], out_hbm.at[0, :, 0], osem.at[b]
            ).wait()

        for b in range(NBUF):
            gather(b, b)

        def group(gi, carry):
            t0 = gi * NBUF
            for b in range(NBUF):
                wait_gather(b)
                transpose(b)
                writeback(t0 + b, b)
            for b in range(NBUF):
                wait_writeback(b)
                gather(t0 + b + NBUF, b)
            return carry

        lax.fori_loop(0, n_groups - 1, group, 0)

        t0 = (n_groups - 1) * NBUF
        for b in range(NBUF):
            wait_gather(b)
            transpose(b)
            writeback(t0 + b, b)
        for b in range(NBUF):
            wait_writeback(b)

    return k


def kernel(attribute, table):
    b0, b1 = attribute.shape
    n = b0 * b1
    info = plsc.get_sparse_core_info()
    nw = info.num_cores * info.num_subcores
    n_blocks = n // BLK
    steps_per_tile = n_blocks // nw
    idx = attribute.T.reshape(n_blocks, BLK).astype(jnp.int32)
    out5 = _make_sc_gather(n_blocks, steps_per_tile, info.num_cores)(idx, table)
    return out5.transpose(2, 4, 0, 1, 3).reshape(b0, b1, D_MODEL)


# speedup vs baseline: 1.6132x; 1.0094x over previous
import functools

import jax
import jax.numpy as jnp
from jax import lax
from jax.experimental import pallas as pl
from jax.experimental.pallas import tpu as pltpu
from jax.experimental.pallas import tpu_sc as plsc

D_MODEL = 64
BLK = 128
NBUF = 4


def _make_sc_gather(n_blocks, steps_per_tile, num_cores):
    mesh = plsc.VectorSubcoreMesh(core_axis_name="c", subcore_axis_name="s")
    n_groups = steps_per_tile // NBUF
    n_j = n_blocks * BLK // 16384
    tiles_i = 16384 // BLK

    @functools.partial(
        pl.kernel,
        mesh=mesh,
        out_type=jax.ShapeDtypeStruct(
            (n_j, D_MODEL // 8, tiles_i, 8, BLK), jnp.float32
        ),
        scratch_types=[
            pltpu.VMEM((steps_per_tile, BLK), jnp.int32),
            pltpu.VMEM((NBUF, BLK, D_MODEL), jnp.float32),
            pltpu.VMEM((NBUF, D_MODEL // 8, 8, BLK), jnp.float32),
            pltpu.SemaphoreType.DMA((NBUF,)),
            pltpu.SemaphoreType.DMA((NBUF,)),
        ],
        compiler_params=pltpu.CompilerParams(
            use_tc_tiling_on_sc=False, needs_layout_passes=False
        ),
    )
    def k(idx_hbm, table_hbm, out_hbm, idx_v, rows_v, blk_v, gsem, osem):
        wid = lax.axis_index("s") * num_cores + lax.axis_index("c")
        row0 = wid * steps_per_tile
        pltpu.sync_copy(idx_hbm.at[pl.ds(row0, steps_per_tile)], idx_v)

        lanes = jnp.arange(16, dtype=jnp.int32)

        def gather(t, b):
            pltpu.async_copy(table_hbm.at[idx_v.at[t]], rows_v.at[b], gsem.at[b])

        def wait_gather(b):
            pltpu.make_async_copy(
                table_hbm.at[pl.ds(0, BLK)], rows_v.at[b], gsem.at[b]
            ).wait()

        def transpose(b):
            @plsc.parallel_loop(0, 32, unroll=4)
            def _(s):
                i0 = (s // 4) * 16
                d0 = (s % 4) * 16
                ivec = lanes + i0
                for k in range(16):
                    dvec = ((lanes + k) & 15) + d0
                    v = plsc.load_gather(rows_v.at[b], [ivec, dvec])
                    plsc.store_scatter(
                        blk_v.at[b], [dvec >> 3, dvec & 7, ivec], v
                    )

        def writeback(t, b):
            n = row0 + t
            j = n // tiles_i
            tc = n % tiles_i
            pltpu.async_copy(blk_v.at[b], out_hbm.at[j, :, tc], osem.at[b])

        def wait_writeback(b):
            pltpu.make_async_copy(
                blk_v.at[b], out_hbm.at[0, :, 0], osem.at[b]
            ).wait()

        for b in range(NBUF):
            gather(b, b)

        def group(gi, carry):
            t0 = gi * NBUF
            for b in range(NBUF):
                wait_gather(b)
                transpose(b)
                writeback(t0 + b, b)
            for b in range(NBUF):
                wait_writeback(b)
                gather(t0 + b + NBUF, b)
            return carry

        lax.fori_loop(0, n_groups - 1, group, 0)

        t0 = (n_groups - 1) * NBUF
        for b in range(NBUF):
            wait_gather(b)
            transpose(b)
            writeback(t0 + b, b)
        for b in range(NBUF):
            wait_writeback(b)

    return k


def kernel(attribute, table):
    b0, b1 = attribute.shape
    n = b0 * b1
    info = plsc.get_sparse_core_info()
    nw = info.num_cores * info.num_subcores
    n_blocks = n // BLK
    steps_per_tile = n_blocks // nw
    idx = attribute.T.reshape(n_blocks, BLK).astype(jnp.int32)
    out5 = _make_sc_gather(n_blocks, steps_per_tile, info.num_cores)(idx, table)
    return out5.transpose(2, 4, 0, 1, 3).reshape(b0, b1, D_MODEL)
